# Initial kernel scaffold; baseline (speedup 1.0000x reference)
import jax
import jax.numpy as jnp
from jax.experimental import pallas as pl


def kernel(atom_features, r, lg_h, params, edge_index, lg_edge_index):
    raise NotImplementedError("write your pallas kernel here")


# speedup vs baseline: 1.0000x; 1.0000x over previous
import jax
import jax.numpy as jnp
from jax.experimental import pallas as pl


def _klin(p, x):
    return x @ p["W"].T + p["b"]


def _kbn(x, p, eps=1e-5):
    mu = jnp.mean(x, axis=0)
    var = jnp.var(x, axis=0)
    return (x - mu) / jnp.sqrt(var + eps) * p["g"] + p["b"]


def _ksilu(x):
    return x * jax.nn.sigmoid(x)


def _kmlp(p, x):
    return _ksilu(_kbn(_klin(p["lin"], x), p["norm"]))


def _krbf(x, vmin, vmax, bins):
    centers = jnp.linspace(vmin, vmax, bins, dtype=jnp.float32)
    gamma = 1.0 / ((vmax - vmin) / (bins - 1))
    return jnp.exp(-gamma * (x[:, None] - centers[None, :]) ** 2)


def _keggc(p, src, dst, node_feats, edge_feats, n_nodes, residual):
    x = _ksilu(_kbn(node_feats, p["norm_nodes"]))
    y = _ksilu(_kbn(edge_feats, p["norm_edges"]))
    e_nodes = _klin(p["src_gate"], x)[src] + _klin(p["dst_gate"], x)[dst]
    y2 = e_nodes + _klin(p["edge_gate"], y)
    sigma = jax.nn.sigmoid(y2)
    Bh = _klin(p["dst_update"], x)
    sum_sigma_h = jax.ops.segment_sum(Bh[src] * sigma, dst, num_segments=n_nodes)
    sum_sigma = jax.ops.segment_sum(sigma, dst, num_segments=n_nodes)
    h = sum_sigma_h / (sum_sigma + 1e-6)
    x_out = _klin(p["src_update"], x) + h
    if residual:
        x_out = node_feats + x_out
        y2 = edge_feats + y2
    return x_out, y2


def kernel(atom_features, r, lg_h, params, edge_index, lg_edge_index):
    src, dst = edge_index[0], edge_index[1]
    lsrc, ldst = lg_edge_index[0], lg_edge_index[1]
    n_nodes = atom_features.shape[0]
    n_edges = r.shape[0]
    z = _krbf(lg_h, -1.0, 1.0, 180)
    z = _kmlp(params["angle_mlp2"], _kmlp(params["angle_mlp1"], z))
    x = _kmlp(params["atom_mlp"], atom_features)
    bondlength = jnp.linalg.norm(r, axis=1)
    y = _krbf(bondlength, 0.0, 8.0, 40)
    y = _kmlp(params["edge_mlp2"], _kmlp(params["edge_mlp1"], y))
    for lp in params["alignn"]:
        x, m = _keggc(lp["node_update"], src, dst, x, y, n_nodes, True)
        y, z = _keggc(lp["edge_update"], lsrc, ldst, m, z, n_edges, True)
    xs = [x]
    ys = [y]
    for gp in params["gcn"]:
        nx, ny = _keggc(gp, src, dst, jnp.concatenate(xs, axis=1), jnp.concatenate(ys, axis=1), n_nodes, False)
        xs.append(nx)
        ys.append(ny)
    x = jnp.concatenate(xs, axis=1)
    h = jnp.mean(x, axis=0, keepdims=True)
    out = _klin(params["fc"], h)
    return jnp.squeeze(out)


# speedup vs baseline: 1.3800x; 1.3800x over previous
import functools

import jax
import jax.numpy as jnp
from jax import lax
from jax.experimental import pallas as pl
from jax.experimental.pallas import tpu as pltpu, tpu_sc as plsc

F32 = jnp.float32
NC, NS = 2, 16
NW = NC * NS


def _klin(p, x):
    return x @ p["W"].T + p["b"]


def _kbn(x, p, eps=1e-5):
    mu = jnp.mean(x, axis=0)
    var = jnp.var(x, axis=0)
    return (x - mu) / jnp.sqrt(var + eps) * p["g"] + p["b"]


def _ksilu(x):
    return x * jax.nn.sigmoid(x)


def _kmlp(p, x):
    return _ksilu(_kbn(_klin(p["lin"], x), p["norm"]))


def _krbf(x, vmin, vmax, bins):
    centers = jnp.linspace(vmin, vmax, bins, dtype=jnp.float32)
    gamma = 1.0 / ((vmax - vmin) / (bins - 1))
    return jnp.exp(-gamma * (x[:, None] - centers[None, :]) ** 2)


def _sc_edge_body(nchunks, C, rows_per_sub,
                  t1_hbm, t2_hbm, ey_hbm, src_hbm, dst_hbm, zeros_hbm,
                  eout_hbm, s_hbm,
                  sidx, didx, t1v, t2v, eyv, msv, acc, sem1, sem2):
    c = lax.axis_index("c")
    s = lax.axis_index("s")
    wid = s * NC + c
    per_w = nchunks * C

    @pl.when(s == 0)
    def _zero():
        pltpu.sync_copy(zeros_hbm, acc)

    plsc.subcore_barrier()

    def chunk(k, _):
        off = wid * per_w + k * C
        pltpu.sync_copy(src_hbm.at[pl.ds(off, C)], sidx)
        pltpu.sync_copy(dst_hbm.at[pl.ds(off, C)], didx)
        g1 = pltpu.async_copy(t1_hbm.at[sidx], t1v, sem1)
        g2 = pltpu.async_copy(t2_hbm.at[didx], t2v, sem2)
        pltpu.sync_copy(ey_hbm.at[pl.ds(off, C)], eyv)
        g1.wait()
        g2.wait()

        def edge(j, _):
            e = t1v[j, 0:16] + t2v[j] + eyv[j]
            t2v[j] = e
            sig = 1.0 / (1.0 + jnp.exp(-e))
            msv[j, 0:16] = t1v[j, 16:32] * sig
            msv[j, 16:32] = sig
            return _

        lax.fori_loop(0, C, edge, None)
        pltpu.sync_copy(t2v, eout_hbm.at[pl.ds(off, C)])
        pltpu.sync_copy(msv, acc.at[didx], add=True)
        return _

    lax.fori_loop(0, nchunks, chunk, None)
    plsc.subcore_barrier()
    n_nodes = zeros_hbm.shape[0]
    tail = n_nodes - (NS - 1) * rows_per_sub
    r0 = s * rows_per_sub

    @pl.when(s < NS - 1)
    def _flush():
        pltpu.sync_copy(acc.at[pl.ds(r0, rows_per_sub)],
                        s_hbm.at[c].at[pl.ds(r0, rows_per_sub)])

    @pl.when(s == NS - 1)
    def _flush_tail():
        pltpu.sync_copy(acc.at[pl.ds((NS - 1) * rows_per_sub, tail)],
                        s_hbm.at[c].at[pl.ds((NS - 1) * rows_per_sub, tail)])


@functools.partial(jax.jit, static_argnames=("n_nodes", "n_edges"))
def _sc_edge_call(t1, t2, ey, src, dst, zeros, n_nodes, n_edges):
    assert n_edges % NW == 0
    per_w = n_edges // NW
    C = 1000
    assert per_w % C == 0
    nchunks = per_w // C
    rows_per_sub = (n_nodes // NS) // 8 * 8
    mesh = plsc.VectorSubcoreMesh(core_axis_name="c", subcore_axis_name="s")
    body = functools.partial(_sc_edge_body, nchunks, C, rows_per_sub)
    f = pl.kernel(
        body,
        out_type=[jax.ShapeDtypeStruct((n_edges, 16), F32),
                  jax.ShapeDtypeStruct((NC, n_nodes, 32), F32)],
        mesh=mesh,
        scratch_types=[
            pltpu.VMEM((C,), jnp.int32),
            pltpu.VMEM((C,), jnp.int32),
            pltpu.VMEM((C, 32), F32),
            pltpu.VMEM((C, 16), F32),
            pltpu.VMEM((C, 16), F32),
            pltpu.VMEM((C, 32), F32),
            pltpu.VMEM_SHARED((n_nodes, 32), F32),
            pltpu.SemaphoreType.DMA,
            pltpu.SemaphoreType.DMA,
        ],
        compiler_params=pltpu.CompilerParams(use_tc_tiling_on_sc=False),
    )
    return f(t1, t2, ey, src, dst, zeros)


def _keggc_sc(p, src, dst, node_feats, edge_feats, n_nodes, residual, zeros):
    n_edges = src.shape[0]
    x = _ksilu(_kbn(node_feats, p["norm_nodes"]))
    y = _ksilu(_kbn(edge_feats, p["norm_edges"]))
    t1 = jnp.concatenate([_klin(p["src_gate"], x), _klin(p["dst_update"], x)], axis=1)
    t2 = _klin(p["dst_gate"], x)
    ey = _klin(p["edge_gate"], y)
    e_out, s_out = _sc_edge_call(t1, t2, ey, src, dst, zeros, n_nodes, n_edges)
    ssum = s_out[0] + s_out[1]
    h = ssum[:, 0:16] / (ssum[:, 16:32] + 1e-6)
    x_out = _klin(p["src_update"], x) + h
    y2 = e_out
    if residual:
        x_out = node_feats + x_out
        y2 = edge_feats + y2
    return x_out, y2


def _keggc(p, src, dst, node_feats, edge_feats, n_nodes, residual):
    x = _ksilu(_kbn(node_feats, p["norm_nodes"]))
    y = _ksilu(_kbn(edge_feats, p["norm_edges"]))
    e_nodes = _klin(p["src_gate"], x)[src] + _klin(p["dst_gate"], x)[dst]
    y2 = e_nodes + _klin(p["edge_gate"], y)
    sigma = jax.nn.sigmoid(y2)
    Bh = _klin(p["dst_update"], x)
    sum_sigma_h = jax.ops.segment_sum(Bh[src] * sigma, dst, num_segments=n_nodes)
    sum_sigma = jax.ops.segment_sum(sigma, dst, num_segments=n_nodes)
    h = sum_sigma_h / (sum_sigma + 1e-6)
    x_out = _klin(p["src_update"], x) + h
    if residual:
        x_out = node_feats + x_out
        y2 = edge_feats + y2
    return x_out, y2


def kernel(atom_features, r, lg_h, params, edge_index, lg_edge_index):
    src, dst = edge_index[0], edge_index[1]
    lsrc, ldst = lg_edge_index[0], lg_edge_index[1]
    n_nodes = atom_features.shape[0]
    n_edges = r.shape[0]
    zeros_n = jnp.zeros((n_nodes, 32), F32)
    z = _krbf(lg_h, -1.0, 1.0, 180)
    z = _kmlp(params["angle_mlp2"], _kmlp(params["angle_mlp1"], z))
    x = _kmlp(params["atom_mlp"], atom_features)
    bondlength = jnp.linalg.norm(r, axis=1)
    y = _krbf(bondlength, 0.0, 8.0, 40)
    y = _kmlp(params["edge_mlp2"], _kmlp(params["edge_mlp1"], y))
    for lp in params["alignn"]:
        x, m = _keggc_sc(lp["node_update"], src, dst, x, y, n_nodes, True, zeros_n)
        y, z = _keggc(lp["edge_update"], lsrc, ldst, m, z, n_edges, True)
    xs = [x]
    ys = [y]
    for gp in params["gcn"]:
        nx, ny = _keggc_sc(gp, src, dst, jnp.concatenate(xs, axis=1),
                           jnp.concatenate(ys, axis=1), n_nodes, False, zeros_n)
        xs.append(nx)
        ys.append(ny)
    x = jnp.concatenate(xs, axis=1)
    h = jnp.mean(x, axis=0, keepdims=True)
    out = _klin(params["fc"], h)
    return jnp.squeeze(out)


# speedup vs baseline: 3.6033x; 2.6110x over previous
import functools

import jax
import jax.numpy as jnp
from jax import lax
from jax.experimental import pallas as pl
from jax.experimental.pallas import tpu as pltpu, tpu_sc as plsc

F32 = jnp.float32
NC, NS = 2, 16
NW = NC * NS


def _klin(p, x):
    return x @ p["W"].T + p["b"]


def _kbn(x, p, eps=1e-5):
    mu = jnp.mean(x, axis=0)
    var = jnp.var(x, axis=0)
    return (x - mu) / jnp.sqrt(var + eps) * p["g"] + p["b"]


def _ksilu(x):
    return x * jax.nn.sigmoid(x)


def _kmlp(p, x):
    return _ksilu(_kbn(_klin(p["lin"], x), p["norm"]))


def _krbf(x, vmin, vmax, bins):
    centers = jnp.linspace(vmin, vmax, bins, dtype=jnp.float32)
    gamma = 1.0 / ((vmax - vmin) / (bins - 1))
    return jnp.exp(-gamma * (x[:, None] - centers[None, :]) ** 2)


def _sc_edge_body(nchunks, C, rows_per_sub,
                  t1_hbm, t2_hbm, ey_hbm, src_hbm, dst_hbm, zeros_hbm,
                  eout_hbm, s_hbm,
                  sidx, didx, t1v, t2v, eyv, msv, acc, sem1, sem2):
    c = lax.axis_index("c")
    s = lax.axis_index("s")
    wid = s * NC + c
    per_w = nchunks * C

    @pl.when(s == 0)
    def _zero():
        pltpu.sync_copy(zeros_hbm, acc)

    plsc.subcore_barrier()

    def chunk(k, _):
        off = wid * per_w + k * C
        pltpu.sync_copy(src_hbm.at[pl.ds(off, C)], sidx)
        pltpu.sync_copy(dst_hbm.at[pl.ds(off, C)], didx)
        g1 = pltpu.async_copy(t1_hbm.at[sidx], t1v, sem1)
        g2 = pltpu.async_copy(t2_hbm.at[didx], t2v, sem2)
        pltpu.sync_copy(ey_hbm.at[pl.ds(off, C)], eyv)
        g1.wait()
        g2.wait()

        def edge(j, _):
            e = t1v[j, 0:16] + t2v[j] + eyv[j]
            t2v[j] = e
            sig = 1.0 / (1.0 + jnp.exp(-e))
            msv[j, 0:16] = t1v[j, 16:32] * sig
            msv[j, 16:32] = sig
            return _

        lax.fori_loop(0, C, edge, None)
        pltpu.sync_copy(t2v, eout_hbm.at[pl.ds(off, C)])
        pltpu.sync_copy(msv, acc.at[didx], add=True)
        return _

    lax.fori_loop(0, nchunks, chunk, None)
    plsc.subcore_barrier()
    n_nodes = zeros_hbm.shape[0]
    tail = n_nodes - (NS - 1) * rows_per_sub
    r0 = s * rows_per_sub

    @pl.when(s < NS - 1)
    def _flush():
        pltpu.sync_copy(acc.at[pl.ds(r0, rows_per_sub)],
                        s_hbm.at[c].at[pl.ds(r0, rows_per_sub)])

    @pl.when(s == NS - 1)
    def _flush_tail():
        pltpu.sync_copy(acc.at[pl.ds((NS - 1) * rows_per_sub, tail)],
                        s_hbm.at[c].at[pl.ds((NS - 1) * rows_per_sub, tail)])


@functools.partial(jax.jit, static_argnames=("n_nodes", "n_edges"))
def _sc_edge_call(t1, t2, ey, src, dst, zeros, n_nodes, n_edges):
    assert n_edges % NW == 0
    per_w = n_edges // NW
    C = 1000
    assert per_w % C == 0
    nchunks = per_w // C
    rows_per_sub = (n_nodes // NS) // 8 * 8
    mesh = plsc.VectorSubcoreMesh(core_axis_name="c", subcore_axis_name="s")
    body = functools.partial(_sc_edge_body, nchunks, C, rows_per_sub)
    f = pl.kernel(
        body,
        out_type=[jax.ShapeDtypeStruct((n_edges, 16), F32),
                  jax.ShapeDtypeStruct((NC, n_nodes, 32), F32)],
        mesh=mesh,
        scratch_types=[
            pltpu.VMEM((C,), jnp.int32),
            pltpu.VMEM((C,), jnp.int32),
            pltpu.VMEM((C, 32), F32),
            pltpu.VMEM((C, 16), F32),
            pltpu.VMEM((C, 16), F32),
            pltpu.VMEM((C, 32), F32),
            pltpu.VMEM_SHARED((n_nodes, 32), F32),
            pltpu.SemaphoreType.DMA,
            pltpu.SemaphoreType.DMA,
        ],
        compiler_params=pltpu.CompilerParams(use_tc_tiling_on_sc=False),
    )
    return f(t1, t2, ey, src, dst, zeros)


def _sc_lgcompute_body(nchunks, C,
                       t1_hbm, t2_hbm, ey_hbm, src_hbm, dst_hbm,
                       eout_hbm, msg_hbm, sig_hbm,
                       sidx, didx, t1v, t2v, eyv, msgv, sigv, sem1, sem2):
    c = lax.axis_index("c")
    s = lax.axis_index("s")
    wid = s * NC + c
    per_w = nchunks * C

    def chunk(k, _):
        off = wid * per_w + k * C
        pltpu.sync_copy(src_hbm.at[pl.ds(off, C)], sidx)
        pltpu.sync_copy(dst_hbm.at[pl.ds(off, C)], didx)
        g1 = pltpu.async_copy(t1_hbm.at[sidx], t1v, sem1)
        g2 = pltpu.async_copy(t2_hbm.at[didx], t2v, sem2)
        pltpu.sync_copy(ey_hbm.at[pl.ds(off, C)], eyv)
        g1.wait()
        g2.wait()

        def edge(j, _):
            e = t1v[j, 0:16] + t2v[j] + eyv[j]
            t2v[j] = e
            sig = 1.0 / (1.0 + jnp.exp(-e))
            msgv[j] = t1v[j, 16:32] * sig
            sigv[j] = sig
            return _

        lax.fori_loop(0, C, edge, None)
        pltpu.sync_copy(t2v, eout_hbm.at[pl.ds(off, C)])
        pltpu.sync_copy(msgv, msg_hbm.at[pl.ds(off, C)])
        pltpu.sync_copy(sigv, sig_hbm.at[pl.ds(off, C)])
        return _

    lax.fori_loop(0, nchunks, chunk, None)


def _sc_lgscatter_body(nchunks, C, R, n_seg,
                       msg_hbm, sig_hbm, pidx_hbm, zeros_hbm,
                       s_hbm,
                       didx, msv, acc, sem1):
    c = lax.axis_index("c")
    s = lax.axis_index("s")
    wid = s * NC + c
    per_w = nchunks * C
    half = n_seg // 2
    rps = half // NS

    for t, val_hbm in enumerate((msg_hbm, sig_hbm)):
        for p in range(2):
            @pl.when(s == 0)
            def _zero():
                pltpu.sync_copy(zeros_hbm, acc)

            plsc.subcore_barrier()

            def chunk(k, _):
                off = wid * per_w + k * C
                pltpu.sync_copy(pidx_hbm.at[p].at[pl.ds(off, C)], didx)
                pltpu.sync_copy(val_hbm.at[pl.ds(off, C)], msv)
                pltpu.sync_copy(msv, acc.at[didx], add=True)
                return _

            lax.fori_loop(0, nchunks, chunk, None)
            plsc.subcore_barrier()
            pltpu.sync_copy(
                acc.at[pl.ds(s * rps, rps)],
                s_hbm.at[c].at[t].at[pl.ds(p * half + s * rps, rps)])
            plsc.subcore_barrier()


@functools.partial(jax.jit, static_argnames=("n_seg", "n_edges"))
def _sc_lg_call(t1, t2, ey, src, dst, pidx, zeros_r, n_seg, n_edges):
    per_w = n_edges // NW
    C = 1000
    nchunks = per_w // C
    mesh = plsc.VectorSubcoreMesh(core_axis_name="c", subcore_axis_name="s")
    R = zeros_r.shape[0]
    fc = pl.kernel(
        functools.partial(_sc_lgcompute_body, nchunks, C),
        out_type=[jax.ShapeDtypeStruct((n_edges, 16), F32),
                  jax.ShapeDtypeStruct((n_edges, 16), F32),
                  jax.ShapeDtypeStruct((n_edges, 16), F32)],
        mesh=mesh,
        scratch_types=[
            pltpu.VMEM((C,), jnp.int32),
            pltpu.VMEM((C,), jnp.int32),
            pltpu.VMEM((C, 32), F32),
            pltpu.VMEM((C, 16), F32),
            pltpu.VMEM((C, 16), F32),
            pltpu.VMEM((C, 16), F32),
            pltpu.VMEM((C, 16), F32),
            pltpu.SemaphoreType.DMA,
            pltpu.SemaphoreType.DMA,
        ],
        compiler_params=pltpu.CompilerParams(use_tc_tiling_on_sc=False),
    )
    e_out, msg, sig = fc(t1, t2, ey, src, dst)
    fs = pl.kernel(
        functools.partial(_sc_lgscatter_body, nchunks, C, R, n_seg),
        out_type=jax.ShapeDtypeStruct((NC, 2, n_seg, 16), F32),
        mesh=mesh,
        scratch_types=[
            pltpu.VMEM((C,), jnp.int32),
            pltpu.VMEM((C, 16), F32),
            pltpu.VMEM_SHARED((R, 16), F32),
            pltpu.SemaphoreType.DMA,
        ],
        compiler_params=pltpu.CompilerParams(use_tc_tiling_on_sc=False),
    )
    s_out = fs(msg, sig, pidx, zeros_r)
    return e_out, s_out


def _keggc_lg_sc(p, src, dst, node_feats, edge_feats, n_seg, pidx, zeros_r):
    n_edges = src.shape[0]
    x = _ksilu(_kbn(node_feats, p["norm_nodes"]))
    y = _ksilu(_kbn(edge_feats, p["norm_edges"]))
    t1 = jnp.concatenate([_klin(p["src_gate"], x), _klin(p["dst_update"], x)], axis=1)
    t2 = _klin(p["dst_gate"], x)
    ey = _klin(p["edge_gate"], y)
    e_out, s_out = _sc_lg_call(t1, t2, ey, src, dst, pidx, zeros_r, n_seg, n_edges)
    ssum = s_out[0] + s_out[1]
    h = ssum[0] / (ssum[1] + 1e-6)
    x_out = node_feats + _klin(p["src_update"], x) + h
    y2 = edge_feats + e_out
    return x_out, y2


def _keggc_sc(p, src, dst, node_feats, edge_feats, n_nodes, residual, zeros):
    n_edges = src.shape[0]
    x = _ksilu(_kbn(node_feats, p["norm_nodes"]))
    y = _ksilu(_kbn(edge_feats, p["norm_edges"]))
    t1 = jnp.concatenate([_klin(p["src_gate"], x), _klin(p["dst_update"], x)], axis=1)
    t2 = _klin(p["dst_gate"], x)
    ey = _klin(p["edge_gate"], y)
    e_out, s_out = _sc_edge_call(t1, t2, ey, src, dst, zeros, n_nodes, n_edges)
    ssum = s_out[0] + s_out[1]
    h = ssum[:, 0:16] / (ssum[:, 16:32] + 1e-6)
    x_out = _klin(p["src_update"], x) + h
    y2 = e_out
    if residual:
        x_out = node_feats + x_out
        y2 = edge_feats + y2
    return x_out, y2


def _keggc(p, src, dst, node_feats, edge_feats, n_nodes, residual):
    x = _ksilu(_kbn(node_feats, p["norm_nodes"]))
    y = _ksilu(_kbn(edge_feats, p["norm_edges"]))
    e_nodes = _klin(p["src_gate"], x)[src] + _klin(p["dst_gate"], x)[dst]
    y2 = e_nodes + _klin(p["edge_gate"], y)
    sigma = jax.nn.sigmoid(y2)
    Bh = _klin(p["dst_update"], x)
    sum_sigma_h = jax.ops.segment_sum(Bh[src] * sigma, dst, num_segments=n_nodes)
    sum_sigma = jax.ops.segment_sum(sigma, dst, num_segments=n_nodes)
    h = sum_sigma_h / (sum_sigma + 1e-6)
    x_out = _klin(p["src_update"], x) + h
    if residual:
        x_out = node_feats + x_out
        y2 = edge_feats + y2
    return x_out, y2


def kernel(atom_features, r, lg_h, params, edge_index, lg_edge_index):
    src, dst = edge_index[0], edge_index[1]
    lsrc, ldst = lg_edge_index[0], lg_edge_index[1]
    n_nodes = atom_features.shape[0]
    n_edges = r.shape[0]
    zeros_n = jnp.zeros((n_nodes, 32), F32)
    half = n_edges // 2
    zeros_r = jnp.zeros((half + 64, 16), F32)
    spread = half + (jnp.arange(ldst.shape[0], dtype=jnp.int32) % 64)
    pidx = jnp.stack([
        jnp.where((ldst >= p * half) & (ldst < (p + 1) * half), ldst - p * half, spread)
        for p in range(2)])
    z = _krbf(lg_h, -1.0, 1.0, 180)
    z = _kmlp(params["angle_mlp2"], _kmlp(params["angle_mlp1"], z))
    x = _kmlp(params["atom_mlp"], atom_features)
    bondlength = jnp.linalg.norm(r, axis=1)
    y = _krbf(bondlength, 0.0, 8.0, 40)
    y = _kmlp(params["edge_mlp2"], _kmlp(params["edge_mlp1"], y))
    for lp in params["alignn"]:
        x, m = _keggc_sc(lp["node_update"], src, dst, x, y, n_nodes, True, zeros_n)
        y, z = _keggc_lg_sc(lp["edge_update"], lsrc, ldst, m, z, n_edges, pidx, zeros_r)
    xs = [x]
    ys = [y]
    for gp in params["gcn"]:
        nx, ny = _keggc_sc(gp, src, dst, jnp.concatenate(xs, axis=1),
                           jnp.concatenate(ys, axis=1), n_nodes, False, zeros_n)
        xs.append(nx)
        ys.append(ny)
    x = jnp.concatenate(xs, axis=1)
    h = jnp.mean(x, axis=0, keepdims=True)
    out = _klin(params["fc"], h)
    return jnp.squeeze(out)
